# initial kernel scaffold (unmeasured)
import jax
import jax.numpy as jnp
from jax import lax
from jax.experimental import pallas as pl
from jax.experimental.pallas import tpu as pltpu

B = 2048
D = 2048


def kernel(partial, gamma):
    def body(p_ref, g_ref, out_ref, send_sem, recv_sem):
        my_x = lax.axis_index("x")
        my_y = lax.axis_index("y")
        my_z = lax.axis_index("z")
        peer = (my_x, 1 - my_y, my_z)

        barrier = pltpu.get_barrier_semaphore()
        pl.semaphore_signal(
            barrier, inc=1, device_id=peer,
            device_id_type=pl.DeviceIdType.MESH,
        )
        pl.semaphore_wait(barrier, 1)

        give_start = (1 - my_y) * B
        rdma = pltpu.make_async_remote_copy(
            src_ref=p_ref.at[0, pl.ds(give_start, B), :],
            dst_ref=out_ref,
            send_sem=send_sem,
            recv_sem=recv_sem,
            device_id=peer,
            device_id_type=pl.DeviceIdType.MESH,
        )
        rdma.start()
        rdma.wait()

        keep_start = my_y * B
        y = out_ref[...] + p_ref[0, pl.ds(keep_start, B), :]
        ms = jnp.mean(y * y, axis=-1, keepdims=True)
        inv = lax.rsqrt(ms + 1e-6)
        out_ref[...] = y * inv * g_ref[...][None, :]

    return pl.pallas_call(
        body,
        out_shape=jax.ShapeDtypeStruct((B, D), jnp.float32),
        in_specs=[
            pl.BlockSpec(memory_space=pltpu.VMEM),
            pl.BlockSpec(memory_space=pltpu.VMEM),
        ],
        out_specs=pl.BlockSpec(memory_space=pltpu.VMEM),
        scratch_shapes=[
            pltpu.SemaphoreType.DMA,
            pltpu.SemaphoreType.DMA,
        ],
        compiler_params=pltpu.CompilerParams(collective_id=0),
    )(partial, gamma)


# baseline (device time: 208076 ns/iter reference)
import jax
import jax.numpy as jnp
from jax import lax
from jax.experimental import pallas as pl
from jax.experimental.pallas import tpu as pltpu

B = 2048
D = 2048


def kernel(partial, gamma):
    def body(p_ref, g_ref, out_ref, keep_ref, send_sem, recv_sem, local_sem):
        my_x = lax.axis_index("x")
        my_y = lax.axis_index("y")
        my_z = lax.axis_index("z")
        peer = (my_x, 1 - my_y, my_z)

        keep_start = my_y * B
        keep_cp = pltpu.make_async_copy(
            p_ref.at[0, pl.ds(keep_start, B), :], keep_ref, local_sem
        )
        keep_cp.start()

        barrier = pltpu.get_barrier_semaphore()
        pl.semaphore_signal(
            barrier, inc=1, device_id=peer,
            device_id_type=pl.DeviceIdType.MESH,
        )
        pl.semaphore_wait(barrier, 1)

        give_start = (1 - my_y) * B
        rdma = pltpu.make_async_remote_copy(
            src_ref=p_ref.at[0, pl.ds(give_start, B), :],
            dst_ref=out_ref,
            send_sem=send_sem,
            recv_sem=recv_sem,
            device_id=peer,
            device_id_type=pl.DeviceIdType.MESH,
        )
        rdma.start()
        rdma.wait()
        keep_cp.wait()

        y = out_ref[...] + keep_ref[...]
        ms = jnp.mean(y * y, axis=-1, keepdims=True)
        inv = lax.rsqrt(ms + 1e-6)
        out_ref[...] = y * inv * g_ref[...][None, :]

    return pl.pallas_call(
        body,
        out_shape=jax.ShapeDtypeStruct((B, D), jnp.float32),
        in_specs=[
            pl.BlockSpec(memory_space=pl.ANY),
            pl.BlockSpec(memory_space=pltpu.VMEM),
        ],
        out_specs=pl.BlockSpec(memory_space=pltpu.VMEM),
        scratch_shapes=[
            pltpu.VMEM((B, D), jnp.float32),
            pltpu.SemaphoreType.DMA,
            pltpu.SemaphoreType.DMA,
            pltpu.SemaphoreType.DMA,
        ],
        compiler_params=pltpu.CompilerParams(
            collective_id=0,
            vmem_limit_bytes=100 * 1024 * 1024,
        ),
    )(partial, gamma)


# device time: 125243 ns/iter; 1.6614x vs baseline; 1.6614x over previous
import jax
import jax.numpy as jnp
from jax import lax
from jax.experimental import pallas as pl
from jax.experimental.pallas import tpu as pltpu

B = 2048
D = 2048
HALF = B // 2
C = 16
CH = HALF // C


def kernel(partial, gamma):
    def body(p_ref, g_ref, out_ref, keep_ref,
             ysend_sems, yrecv_sems, zsend_sems, zrecv_sems, local_sem):
        my_x = lax.axis_index("x")
        my_y = lax.axis_index("y")
        my_z = lax.axis_index("z")
        peer_y = (my_x, 1 - my_y, my_z)
        peer_z = (my_x, my_y, 1 - my_z)

        keep_cp = pltpu.make_async_copy(
            p_ref.at[0, pl.ds(my_y * B, B), :], keep_ref, local_sem
        )
        keep_cp.start()

        barrier = pltpu.get_barrier_semaphore()
        for nbr in (peer_y, peer_z):
            pl.semaphore_signal(
                barrier, inc=1, device_id=nbr,
                device_id_type=pl.DeviceIdType.MESH,
            )
        pl.semaphore_wait(barrier, 2)

        give0 = (1 - my_y) * B + my_z * HALF
        dst0 = my_z * HALF
        y_rdmas = []
        for k in range(C):
            r = pltpu.make_async_remote_copy(
                src_ref=p_ref.at[0, pl.ds(give0 + k * CH, CH), :],
                dst_ref=out_ref.at[pl.ds(dst0 + k * CH, CH), :],
                send_sem=ysend_sems.at[k],
                recv_sem=yrecv_sems.at[k],
                device_id=peer_y,
                device_id_type=pl.DeviceIdType.MESH,
            )
            r.start()
            y_rdmas.append(r)

        z_rdmas = []
        for k in range(C):
            y_rdmas[k].wait_recv()
            zr = pltpu.make_async_remote_copy(
                src_ref=out_ref.at[pl.ds(dst0 + k * CH, CH), :],
                dst_ref=out_ref.at[pl.ds(dst0 + k * CH, CH), :],
                send_sem=zsend_sems.at[k],
                recv_sem=zrecv_sems.at[k],
                device_id=peer_z,
                device_id_type=pl.DeviceIdType.MESH,
            )
            zr.start()
            z_rdmas.append(zr)

        zr0 = (1 - my_z) * HALF
        zrecv_descs = []
        for k in range(C):
            zrecv_descs.append(pltpu.make_async_remote_copy(
                src_ref=out_ref.at[pl.ds(zr0 + k * CH, CH), :],
                dst_ref=out_ref.at[pl.ds(zr0 + k * CH, CH), :],
                send_sem=zsend_sems.at[k],
                recv_sem=zrecv_sems.at[k],
                device_id=peer_z,
                device_id_type=pl.DeviceIdType.MESH,
            ))

        keep_cp.wait()
        g = g_ref[...]

        def norm_chunk(r0):
            yrow = out_ref[pl.ds(r0, CH), :] + keep_ref[pl.ds(r0, CH), :]
            ms = jnp.mean(yrow * yrow, axis=-1, keepdims=True)
            out_ref[pl.ds(r0, CH), :] = yrow * lax.rsqrt(ms + 1e-6) * g[None, :]

        for k in range(C):
            z_rdmas[k].wait_send()
            norm_chunk(dst0 + k * CH)

        for k in range(C):
            zrecv_descs[k].wait_recv()
            norm_chunk(zr0 + k * CH)

        for k in range(C):
            y_rdmas[k].wait_send()

    return pl.pallas_call(
        body,
        out_shape=jax.ShapeDtypeStruct((B, D), jnp.float32),
        in_specs=[
            pl.BlockSpec(memory_space=pl.ANY),
            pl.BlockSpec(memory_space=pltpu.VMEM),
        ],
        out_specs=pl.BlockSpec(memory_space=pltpu.VMEM),
        scratch_shapes=[
            pltpu.VMEM((B, D), jnp.float32),
            pltpu.SemaphoreType.DMA((C,)),
            pltpu.SemaphoreType.DMA((C,)),
            pltpu.SemaphoreType.DMA((C,)),
            pltpu.SemaphoreType.DMA((C,)),
            pltpu.SemaphoreType.DMA,
        ],
        compiler_params=pltpu.CompilerParams(
            collective_id=0,
            vmem_limit_bytes=100 * 1024 * 1024,
        ),
    )(partial, gamma)


# device time: 122746 ns/iter; 1.6952x vs baseline; 1.0203x over previous
import jax
import jax.numpy as jnp
from jax import lax
from jax.experimental import pallas as pl
from jax.experimental.pallas import tpu as pltpu

B = 2048
D = 2048
HALF = B // 2
C = 16
CH = HALF // C


def kernel(partial, gamma):
    def body(p_ref, g_ref, out_ref, keep_ref,
             ysend_sems, yrecv_sems, zsend_sems, zrecv_sems, local_sem):
        my_x = lax.axis_index("x")
        my_y = lax.axis_index("y")
        my_z = lax.axis_index("z")
        peer_y = (my_x, 1 - my_y, my_z)
        peer_z = (my_x, my_y, 1 - my_z)

        keep_cp = pltpu.make_async_copy(
            p_ref.at[0, pl.ds(my_y * B, B), :], keep_ref, local_sem
        )
        keep_cp.start()

        barrier = pltpu.get_barrier_semaphore()
        for nbr in (peer_y, peer_z):
            pl.semaphore_signal(
                barrier, inc=1, device_id=nbr,
                device_id_type=pl.DeviceIdType.MESH,
            )
        pl.semaphore_wait(barrier, 2)

        give0 = (1 - my_y) * B + my_z * HALF
        dst0 = my_z * HALF
        y_rdmas = []
        for k in range(C):
            r = pltpu.make_async_remote_copy(
                src_ref=p_ref.at[0, pl.ds(give0 + k * CH, CH), :],
                dst_ref=out_ref.at[pl.ds(dst0 + k * CH, CH), :],
                send_sem=ysend_sems.at[k],
                recv_sem=yrecv_sems.at[k],
                device_id=peer_y,
                device_id_type=pl.DeviceIdType.MESH,
            )
            r.start()
            y_rdmas.append(r)

        z_rdmas = []
        for k in range(C):
            y_rdmas[k].wait_recv()
            zr = pltpu.make_async_remote_copy(
                src_ref=out_ref.at[pl.ds(dst0 + k * CH, CH), :],
                dst_ref=out_ref.at[pl.ds(dst0 + k * CH, CH), :],
                send_sem=zsend_sems.at[k],
                recv_sem=zrecv_sems.at[k],
                device_id=peer_z,
                device_id_type=pl.DeviceIdType.MESH,
            )
            zr.start()
            z_rdmas.append(zr)

        zr0 = (1 - my_z) * HALF
        zrecv_descs = []
        for k in range(C):
            zrecv_descs.append(pltpu.make_async_remote_copy(
                src_ref=out_ref.at[pl.ds(zr0 + k * CH, CH), :],
                dst_ref=out_ref.at[pl.ds(zr0 + k * CH, CH), :],
                send_sem=zsend_sems.at[k],
                recv_sem=zrecv_sems.at[k],
                device_id=peer_z,
                device_id_type=pl.DeviceIdType.MESH,
            ))

        keep_cp.wait()
        g = g_ref[...]

        def norm_chunk(r0):
            yrow = out_ref[pl.ds(r0, CH), :] + keep_ref[pl.ds(r0, CH), :]
            ms = jnp.mean(yrow * yrow, axis=-1, keepdims=True)
            out_ref[pl.ds(r0, CH), :] = yrow * lax.rsqrt(ms + 1e-6) * g[None, :]

        for k in range(C):
            z_rdmas[k].wait_send()
            norm_chunk(dst0 + k * CH)
            zrecv_descs[k].wait_recv()
            norm_chunk(zr0 + k * CH)

        for k in range(C):
            y_rdmas[k].wait_send()

    return pl.pallas_call(
        body,
        out_shape=jax.ShapeDtypeStruct((B, D), jnp.float32),
        in_specs=[
            pl.BlockSpec(memory_space=pl.ANY),
            pl.BlockSpec(memory_space=pltpu.VMEM),
        ],
        out_specs=pl.BlockSpec(memory_space=pltpu.VMEM),
        scratch_shapes=[
            pltpu.VMEM((B, D), jnp.float32),
            pltpu.SemaphoreType.DMA((C,)),
            pltpu.SemaphoreType.DMA((C,)),
            pltpu.SemaphoreType.DMA((C,)),
            pltpu.SemaphoreType.DMA((C,)),
            pltpu.SemaphoreType.DMA,
        ],
        compiler_params=pltpu.CompilerParams(
            collective_id=0,
            vmem_limit_bytes=100 * 1024 * 1024,
        ),
    )(partial, gamma)


# device time: 98967 ns/iter; 2.1025x vs baseline; 1.2403x over previous
import os

import jax
import jax.numpy as jnp
from jax import lax
from jax.experimental import pallas as pl
from jax.experimental.pallas import tpu as pltpu

B = 2048
D = 2048
Q = 512
SUB = 256
CH = int(os.environ.get("KERNEL_CH", "32"))
CS = SUB // CH
CQ = Q // CH
_DO_NORM = os.environ.get("KERNEL_SKIP_NORM", "0") != "1"


def kernel(partial, gamma):
    def body(p_ref, g_ref, out_ref, wbuf_ref, keep_ref,
             ys_sems, yr_sems, p1xs, p1xr, p1zs, p1zr,
             p2xs, p2xr, p2zs, p2zr, local_sem, wb_sems):
        my_x = lax.axis_index("x")
        my_y = lax.axis_index("y")
        my_z = lax.axis_index("z")
        peer_y = (my_x, 1 - my_y, my_z)
        peer_x = (1 - my_x, my_y, my_z)
        peer_z = (my_x, my_y, 1 - my_z)

        q = 2 * my_x + my_z
        qx = 2 * (1 - my_x) + my_z
        qz = 2 * my_x + (1 - my_z)
        qd = 2 * (1 - my_x) + (1 - my_z)
        aq, bq = Q * q, Q * q + SUB
        aqx, bqx = Q * qx, Q * qx + SUB
        aqz, bqz = Q * qz, Q * qz + SUB
        aqd, bqd = Q * qd, Q * qd + SUB

        keep_cp = pltpu.make_async_copy(
            p_ref.at[0, pl.ds(my_y * B, B), :], keep_ref, local_sem
        )
        keep_cp.start()

        barrier = pltpu.get_barrier_semaphore()
        for nbr in (peer_y, peer_x, peer_z):
            pl.semaphore_signal(
                barrier, inc=1, device_id=nbr,
                device_id_type=pl.DeviceIdType.MESH,
            )
        pl.semaphore_wait(barrier, 3)

        def face_send(r0, ssem, rsem, dev):
            r = pltpu.make_async_remote_copy(
                src_ref=wbuf_ref.at[pl.ds(r0, CH), :],
                dst_ref=wbuf_ref.at[pl.ds(r0, CH), :],
                send_sem=ssem, recv_sem=rsem,
                device_id=dev, device_id_type=pl.DeviceIdType.MESH,
            )
            r.start()
            return r

        def face_recv(r0, ssem, rsem, dev):
            return pltpu.make_async_remote_copy(
                src_ref=wbuf_ref.at[pl.ds(r0, CH), :],
                dst_ref=wbuf_ref.at[pl.ds(r0, CH), :],
                send_sem=ssem, recv_sem=rsem,
                device_id=dev, device_id_type=pl.DeviceIdType.MESH,
            )

        give0 = (1 - my_y) * B + Q * q
        y_rdmas = []
        for k in range(CQ):
            r = pltpu.make_async_remote_copy(
                src_ref=p_ref.at[0, pl.ds(give0 + k * CH, CH), :],
                dst_ref=wbuf_ref.at[pl.ds(Q * q + k * CH, CH), :],
                send_sem=ys_sems.at[k], recv_sem=yr_sems.at[k],
                device_id=peer_y, device_id_type=pl.DeviceIdType.MESH,
            )
            r.start()
            y_rdmas.append(r)

        keep_cp.wait()
        g = g_ref[...]
        sends = list(y_rdmas)
        wb_cps = []

        def norm_wb(r0):
            if _DO_NORM:
                yrow = wbuf_ref[pl.ds(r0, CH), :] + keep_ref[pl.ds(r0, CH), :]
                ms = jnp.mean(yrow * yrow, axis=-1, keepdims=True)
                keep_ref[pl.ds(r0, CH), :] = (
                    yrow * lax.rsqrt(ms + 1e-6) * g[None, :]
                )
            cp = pltpu.make_async_copy(
                keep_ref.at[pl.ds(r0, CH), :],
                out_ref.at[pl.ds(r0, CH), :],
                wb_sems.at[len(wb_cps)],
            )
            cp.start()
            wb_cps.append(cp)

        for k in range(CS):
            y_rdmas[k].wait_recv()
            sends.append(face_send(aq + k * CH, p1xs.at[k], p1xr.at[k], peer_x))
            sends.append(face_send(aq + k * CH, p2zs.at[k], p2zr.at[k], peer_z))
            norm_wb(aq + k * CH)

        for k in range(CS):
            face_recv(aqx + k * CH, p1xs.at[k], p1xr.at[k], peer_x).wait_recv()
            sends.append(face_send(aqx + k * CH, p2zs.at[CS + k],
                                   p2zr.at[CS + k], peer_z))
            norm_wb(aqx + k * CH)

            y_rdmas[CS + k].wait_recv()
            sends.append(face_send(bq + k * CH, p1zs.at[k], p1zr.at[k], peer_z))
            sends.append(face_send(bq + k * CH, p2xs.at[k], p2xr.at[k], peer_x))
            norm_wb(bq + k * CH)

        for k in range(CS):
            face_recv(bqz + k * CH, p1zs.at[k], p1zr.at[k], peer_z).wait_recv()
            sends.append(face_send(bqz + k * CH, p2xs.at[CS + k],
                                   p2xr.at[CS + k], peer_x))
            norm_wb(bqz + k * CH)

        for j in range(CQ):
            r0 = aqz + j * CH if j < CS else aqd + (j - CS) * CH
            face_recv(r0, p2zs.at[j], p2zr.at[j], peer_z).wait_recv()
            norm_wb(r0)
        for j in range(CQ):
            r0 = bqx + j * CH if j < CS else bqd + (j - CS) * CH
            face_recv(r0, p2xs.at[j], p2xr.at[j], peer_x).wait_recv()
            norm_wb(r0)

        for s in sends:
            s.wait_send()
        for cp in wb_cps:
            cp.wait()

    n_wb = (B // CH)
    return pl.pallas_call(
        body,
        out_shape=jax.ShapeDtypeStruct((B, D), jnp.float32),
        in_specs=[
            pl.BlockSpec(memory_space=pl.ANY),
            pl.BlockSpec(memory_space=pltpu.VMEM),
        ],
        out_specs=pl.BlockSpec(memory_space=pl.ANY),
        scratch_shapes=[
            pltpu.VMEM((B, D), jnp.float32),
            pltpu.VMEM((B, D), jnp.float32),
            pltpu.SemaphoreType.DMA((CQ,)),
            pltpu.SemaphoreType.DMA((CQ,)),
            pltpu.SemaphoreType.DMA((CS,)),
            pltpu.SemaphoreType.DMA((CS,)),
            pltpu.SemaphoreType.DMA((CS,)),
            pltpu.SemaphoreType.DMA((CS,)),
            pltpu.SemaphoreType.DMA((CQ,)),
            pltpu.SemaphoreType.DMA((CQ,)),
            pltpu.SemaphoreType.DMA((CQ,)),
            pltpu.SemaphoreType.DMA((CQ,)),
            pltpu.SemaphoreType.DMA,
            pltpu.SemaphoreType.DMA((n_wb,)),
        ],
        compiler_params=pltpu.CompilerParams(
            collective_id=0,
            vmem_limit_bytes=100 * 1024 * 1024,
        ),
    )(partial, gamma)
